# Initial kernel scaffold; baseline (speedup 1.0000x reference)
import jax
import jax.numpy as jnp
from jax.experimental import pallas as pl


def kernel(x, edge_index, edge_attr, W1, a_src1, a_dst1, b1, W2, a_src2, a_dst2, b2, W3, a_src3, a_dst3, b3, Wm1, bm1, Wm2, bm2, Wm3, bm3):
    raise NotImplementedError("write your pallas kernel here")


# speedup vs baseline: 1.4011x; 1.4011x over previous
import jax
import jax.numpy as jnp
from jax.experimental import pallas as pl


N_NODES = 10000
E_EDGES = 160000
MLP_BLK = 3200


def _mlp_body(ef_ref, ea_ref, w1_ref, b1_ref, w2_ref, b2_ref, w3_ref, b3_ref, out_ref):
    ef = ef_ref[...]
    z = jnp.maximum(ef @ w1_ref[...] + b1_ref[...], 0.0)
    z = jnp.maximum(z @ w2_ref[...] + b2_ref[...], 0.0)
    z = z @ w3_ref[...] + b3_ref[...]
    out_ref[...] = ea_ref[...] + z


def _edge_mlp(ef, ea, Wm1, bm1, Wm2, bm2, Wm3, bm3):
    E = ef.shape[0]
    hid = Wm1.shape[1]
    edim = Wm3.shape[1]
    grid = (E // MLP_BLK,)
    return pl.pallas_call(
        _mlp_body,
        grid=grid,
        in_specs=[
            pl.BlockSpec((MLP_BLK, ef.shape[1]), lambda i: (i, 0)),
            pl.BlockSpec((MLP_BLK, edim), lambda i: (i, 0)),
            pl.BlockSpec(Wm1.shape, lambda i: (0, 0)),
            pl.BlockSpec((1, hid), lambda i: (0, 0)),
            pl.BlockSpec(Wm2.shape, lambda i: (0, 0)),
            pl.BlockSpec((1, hid), lambda i: (0, 0)),
            pl.BlockSpec(Wm3.shape, lambda i: (0, 0)),
            pl.BlockSpec((1, edim), lambda i: (0, 0)),
        ],
        out_specs=pl.BlockSpec((MLP_BLK, edim), lambda i: (i, 0)),
        out_shape=jax.ShapeDtypeStruct((E, edim), jnp.float32),
    )(ef, ea, Wm1, bm1.reshape(1, -1), Wm2, bm2.reshape(1, -1), Wm3, bm3.reshape(1, -1))


def _gat(x, src, dst, N, W, a_src, a_dst, b):
    h = x @ W
    ts = h @ a_src
    td = h @ a_dst
    e = ts[src] + td[dst]
    e = jnp.maximum(e, 0.2 * e)
    ex = jnp.exp(e)
    denom = jax.ops.segment_sum(ex, dst, num_segments=N)
    num = jax.ops.segment_sum(ex[:, None] * h[src], dst, num_segments=N)
    return num / (denom[:, None] + 1e-16) + b


def kernel(x, edge_index, edge_attr, W1, a_src1, a_dst1, b1, W2, a_src2, a_dst2, b2, W3, a_src3, a_dst3, b3, Wm1, bm1, Wm2, bm2, Wm3, bm3):
    N = x.shape[0]
    row, col = edge_index[0], edge_index[1]
    loop = jnp.arange(N)
    src = jnp.concatenate([row, loop])
    dst = jnp.concatenate([col, loop])
    h = _gat(x, src, dst, N, W1, a_src1, a_dst1, b1)
    h = _gat(h, src, dst, N, W2, a_src2, a_dst2, b2)
    h = _gat(h, src, dst, N, W3, a_src3, a_dst3, b3)
    ef = jnp.concatenate([h[row], h[col], edge_attr], axis=-1)
    return _edge_mlp(ef, edge_attr, Wm1, bm1, Wm2, bm2, Wm3, bm3)


# speedup vs baseline: 14.9167x; 10.6462x over previous
import functools

import jax
import jax.numpy as jnp
from jax import lax
from jax.experimental import pallas as pl
from jax.experimental.pallas import tpu as pltpu
from jax.experimental.pallas import tpu_sc as plsc


N_NODES = 10000
NP = 10240
E_EDGES = 160000
EP = 163840
PW = EP // 32
K = 128
NCH = PW // K
RPS = NP // 16
MLP_BLK = 3200
TC_BLK = 1280

_SC_MESH = plsc.VectorSubcoreMesh(core_axis_name="c", subcore_axis_name="s",
                                  num_cores=2, num_subcores=16)


def _tc_stage_body(x_ref, w_ref, asrc_ref, adst_ref,
                   h_ref, ts_ref, td_ref, wself_ref, sacc_ref):
    h = x_ref[...] @ w_ref[...]
    ts = h @ asrc_ref[...]
    td = h @ adst_ref[...]
    e = ts + td
    ws = jnp.exp(jnp.maximum(e, 0.2 * e))
    h_ref[...] = h
    ts_ref[...] = ts
    td_ref[...] = td
    wself_ref[...] = ws
    sacc_ref[...] = h * ws


def _tc_stage(x, W, a_src, a_dst):
    cin, c = W.shape
    grid = (NP // TC_BLK,)
    return pl.pallas_call(
        _tc_stage_body,
        grid=grid,
        in_specs=[
            pl.BlockSpec((TC_BLK, cin), lambda i: (i, 0)),
            pl.BlockSpec((cin, c), lambda i: (0, 0)),
            pl.BlockSpec((c, 1), lambda i: (0, 0)),
            pl.BlockSpec((c, 1), lambda i: (0, 0)),
        ],
        out_specs=[
            pl.BlockSpec((TC_BLK, c), lambda i: (i, 0)),
            pl.BlockSpec((TC_BLK, 1), lambda i: (i, 0)),
            pl.BlockSpec((TC_BLK, 1), lambda i: (i, 0)),
            pl.BlockSpec((TC_BLK, 1), lambda i: (i, 0)),
            pl.BlockSpec((TC_BLK, c), lambda i: (i, 0)),
        ],
        out_shape=[
            jax.ShapeDtypeStruct((NP, c), jnp.float32),
            jax.ShapeDtypeStruct((NP, 1), jnp.float32),
            jax.ShapeDtypeStruct((NP, 1), jnp.float32),
            jax.ShapeDtypeStruct((NP, 1), jnp.float32),
            jax.ShapeDtypeStruct((NP, c), jnp.float32),
        ],
    )(x, W, a_src.reshape(-1, 1), a_dst.reshape(-1, 1))


def _tc_combine_stage_body(al_ref, ar_ref, d_ref, b_ref,
                           w_ref, asrc_ref, adst_ref,
                           h_ref, ts_ref, td_ref, wself_ref, sacc_ref):
    acc = jnp.concatenate([al_ref[...], ar_ref[...]], axis=-1)
    x = acc / (d_ref[...] + 1e-16) + b_ref[...]
    h = x @ w_ref[...]
    ts = h @ asrc_ref[...]
    td = h @ adst_ref[...]
    e = ts + td
    ws = jnp.exp(jnp.maximum(e, 0.2 * e))
    h_ref[...] = h
    ts_ref[...] = ts
    td_ref[...] = td
    wself_ref[...] = ws
    sacc_ref[...] = h * ws


def _tc_combine_stage(acc_cat, den, b, W, a_src, a_dst):
    cin, c = W.shape
    hcin = cin // 2
    nb = NP // TC_BLK
    grid = (nb,)
    return pl.pallas_call(
        _tc_combine_stage_body,
        grid=grid,
        in_specs=[
            pl.BlockSpec((TC_BLK, hcin), lambda i: (i, 0)),
            pl.BlockSpec((TC_BLK, hcin), lambda i: (i + NP // TC_BLK, 0)),
            pl.BlockSpec((TC_BLK, 1), lambda i: (i, 0)),
            pl.BlockSpec((1, cin), lambda i: (0, 0)),
            pl.BlockSpec((cin, c), lambda i: (0, 0)),
            pl.BlockSpec((c, 1), lambda i: (0, 0)),
            pl.BlockSpec((c, 1), lambda i: (0, 0)),
        ],
        out_specs=[
            pl.BlockSpec((TC_BLK, c), lambda i: (i, 0)),
            pl.BlockSpec((TC_BLK, 1), lambda i: (i, 0)),
            pl.BlockSpec((TC_BLK, 1), lambda i: (i, 0)),
            pl.BlockSpec((TC_BLK, 1), lambda i: (i, 0)),
            pl.BlockSpec((TC_BLK, c), lambda i: (i, 0)),
        ],
        out_shape=[
            jax.ShapeDtypeStruct((NP, c), jnp.float32),
            jax.ShapeDtypeStruct((NP, 1), jnp.float32),
            jax.ShapeDtypeStruct((NP, 1), jnp.float32),
            jax.ShapeDtypeStruct((NP, 1), jnp.float32),
            jax.ShapeDtypeStruct((NP, c), jnp.float32),
        ],
    )(acc_cat, acc_cat, den.reshape(-1, 1), b.reshape(1, -1), W,
      a_src.reshape(-1, 1), a_dst.reshape(-1, 1))


def _tc_combine_body(a0_ref, a1_ref, d0_ref, d1_ref, b_ref, out_ref):
    out_ref[...] = ((a0_ref[...] + a1_ref[...])
                    / (d0_ref[...] + d1_ref[...] + 1e-16) + b_ref[...])


def _tc_combine(acc_cat, den_cat, b):
    c = acc_cat.shape[1]
    grid = (NP // TC_BLK,)
    return pl.pallas_call(
        _tc_combine_body,
        grid=grid,
        in_specs=[
            pl.BlockSpec((TC_BLK, c), lambda i: (i, 0)),
            pl.BlockSpec((TC_BLK, c), lambda i: (i + NP // TC_BLK, 0)),
            pl.BlockSpec((TC_BLK, 1), lambda i: (i, 0)),
            pl.BlockSpec((TC_BLK, 1), lambda i: (i + NP // TC_BLK, 0)),
            pl.BlockSpec((1, c), lambda i: (0, 0)),
        ],
        out_specs=pl.BlockSpec((TC_BLK, c), lambda i: (i, 0)),
        out_shape=jax.ShapeDtypeStruct((NP, c), jnp.float32),
    )(acc_cat, acc_cat, den_cat.reshape(-1, 1), den_cat.reshape(-1, 1),
      b.reshape(1, -1))


def _iota16():
    return lax.iota(jnp.int32, 16)


NCH2 = EP // 16 // K


def _make_sc_aggregate(c):
    hc = c // 2

    def body(hcat_hbm, ts_hbm, td_hbm, src_hbm, dst_hbm,
             init_hbm, wself_hbm,
             acc_out, den_out,
             ts_v, td_v, src_v, dst_v, w_v, rows0, rows1,
             acc_sh, den_sh, sem0, sem1):
        cid = lax.axis_index("c")
        sid = lax.axis_index("s")
        r0 = sid * RPS
        pltpu.sync_copy(init_hbm.at[pl.ds(cid * NP + r0, RPS)],
                        acc_sh.at[pl.ds(r0, RPS)])
        pltpu.sync_copy(wself_hbm.at[pl.ds(r0, RPS)],
                        den_sh.at[pl.ds(r0, RPS)])

        pltpu.sync_copy(ts_hbm, ts_v)
        pltpu.sync_copy(td_hbm, td_v)
        pltpu.sync_copy(src_hbm.at[pl.ds(sid * NCH2, NCH2)], src_v)
        pltpu.sync_copy(dst_hbm.at[pl.ds(sid * NCH2, NCH2)], dst_v)

        off = jnp.full((16,), cid * NP, jnp.int32)

        def wchunk(i, _):
            for j in range(K // 16):
                sl = pl.ds(j * 16, 16)
                sidx = src_v[i, sl]
                didx = dst_v[i, sl]
                e = (plsc.load_gather(ts_v, [sidx])
                     + plsc.load_gather(td_v, [didx]))
                w_v[i, sl] = jnp.exp(jnp.maximum(e, 0.2 * e))
                src_v[i, sl] = sidx + off
            return 0

        lax.fori_loop(0, NCH2, wchunk, 0)
        plsc.subcore_barrier()

        pltpu.async_copy(hcat_hbm.at[src_v.at[0]], rows0, sem0)

        def chunk(i, cur, csem, pf, nxt, nsem):
            pltpu.make_async_copy(hcat_hbm.at[src_v.at[i]], cur, csem).wait()

            @pl.when(pf < NCH2)
            def _():
                pltpu.async_copy(hcat_hbm.at[src_v.at[pf]], nxt, nsem)

            def mul_row(e2, _):
                wb = plsc.load_gather(
                    w_v, [jnp.full((16,), i, jnp.int32),
                          jnp.full((16,), e2, jnp.int32)])
                for jj in range(hc // 16):
                    sl2 = pl.ds(jj * 16, 16)
                    cur[e2, sl2] = cur[e2, sl2] * wb
                return 0

            lax.fori_loop(0, K, mul_row, 0)
            pltpu.sync_copy(cur, acc_sh.at[dst_v.at[i]], add=True)
            pltpu.sync_copy(w_v.at[i], den_sh.at[dst_v.at[i]], add=True)

        def pair(p, _):
            i0 = p * 2
            chunk(i0, rows0, sem0, i0 + 1, rows1, sem1)
            chunk(i0 + 1, rows1, sem1, i0 + 2, rows0, sem0)
            return 0

        lax.fori_loop(0, NCH2 // 2, pair, 0)
        plsc.subcore_barrier()
        pltpu.sync_copy(acc_sh.at[pl.ds(r0, RPS)],
                        acc_out.at[pl.ds(cid * NP + r0, RPS)])
        pltpu.sync_copy(den_sh.at[pl.ds(r0, RPS)],
                        den_out.at[pl.ds(cid * NP + r0, RPS)])

    return pl.kernel(
        body,
        out_type=[
            jax.ShapeDtypeStruct((2 * NP, hc), jnp.float32),
            jax.ShapeDtypeStruct((2 * NP,), jnp.float32),
        ],
        mesh=_SC_MESH,
        compiler_params=pltpu.CompilerParams(needs_layout_passes=False, use_tc_tiling_on_sc=False),
        scratch_types=[
            pltpu.VMEM((NP,), jnp.float32),
            pltpu.VMEM((NP,), jnp.float32),
            pltpu.VMEM((NCH2, K), jnp.int32),
            pltpu.VMEM((NCH2, K), jnp.int32),
            pltpu.VMEM((NCH2, K), jnp.float32),
            pltpu.VMEM((K, hc), jnp.float32),
            pltpu.VMEM((K, hc), jnp.float32),
            pltpu.VMEM_SHARED((NP, hc), jnp.float32),
            pltpu.VMEM_SHARED((NP,), jnp.float32),
            pltpu.SemaphoreType.DMA,
            pltpu.SemaphoreType.DMA,
        ],
    )


def _make_sc_aggregate_es(c):

    def body(h_hbm, ts_hbm, td_hbm, src_hbm, dst_hbm,
             initcat_hbm, wscat_hbm,
             acc_out, den_out,
             ts_v, td_v, src_v, dst_v, w_v, rows0, rows1,
             acc_sh, den_sh, sem0, sem1):
        cid = lax.axis_index("c")
        sid = lax.axis_index("s")
        wid = cid * 16 + sid
        r0 = sid * RPS
        pltpu.sync_copy(initcat_hbm.at[pl.ds(cid * NP + r0, RPS)],
                        acc_sh.at[pl.ds(r0, RPS)])
        pltpu.sync_copy(wscat_hbm.at[pl.ds(cid * NP + r0, RPS)],
                        den_sh.at[pl.ds(r0, RPS)])

        pltpu.sync_copy(ts_hbm, ts_v)
        pltpu.sync_copy(td_hbm, td_v)
        pltpu.sync_copy(src_hbm.at[pl.ds(wid * NCH, NCH)], src_v)
        pltpu.sync_copy(dst_hbm.at[pl.ds(wid * NCH, NCH)], dst_v)

        def wchunk(i, _):
            for j in range(K // 16):
                sl = pl.ds(j * 16, 16)
                sidx = src_v[i, sl]
                didx = dst_v[i, sl]
                e = (plsc.load_gather(ts_v, [sidx])
                     + plsc.load_gather(td_v, [didx]))
                w_v[i, sl] = jnp.exp(jnp.maximum(e, 0.2 * e))
            return 0

        lax.fori_loop(0, NCH, wchunk, 0)
        plsc.subcore_barrier()

        pltpu.async_copy(h_hbm.at[src_v.at[0]], rows0, sem0)

        def chunk(i, cur, csem, pf, nxt, nsem):
            pltpu.make_async_copy(h_hbm.at[src_v.at[i]], cur, csem).wait()

            @pl.when(pf < NCH)
            def _():
                pltpu.async_copy(h_hbm.at[src_v.at[pf]], nxt, nsem)

            def mul_row(e2, _):
                wb = plsc.load_gather(
                    w_v, [jnp.full((16,), i, jnp.int32),
                          jnp.full((16,), e2, jnp.int32)])
                for jj in range(c // 16):
                    sl2 = pl.ds(jj * 16, 16)
                    cur[e2, sl2] = cur[e2, sl2] * wb
                return 0

            lax.fori_loop(0, K, mul_row, 0)
            pltpu.sync_copy(cur, acc_sh.at[dst_v.at[i]], add=True)
            pltpu.sync_copy(w_v.at[i], den_sh.at[dst_v.at[i]], add=True)

        def pair(p, _):
            i0 = p * 2
            chunk(i0, rows0, sem0, i0 + 1, rows1, sem1)
            chunk(i0 + 1, rows1, sem1, i0 + 2, rows0, sem0)
            return 0

        lax.fori_loop(0, NCH // 2, pair, 0)
        plsc.subcore_barrier()
        pltpu.sync_copy(acc_sh.at[pl.ds(r0, RPS)],
                        acc_out.at[pl.ds(cid * NP + r0, RPS)])
        pltpu.sync_copy(den_sh.at[pl.ds(r0, RPS)],
                        den_out.at[pl.ds(cid * NP + r0, RPS)])

    return pl.kernel(
        body,
        out_type=[
            jax.ShapeDtypeStruct((2 * NP, c), jnp.float32),
            jax.ShapeDtypeStruct((2 * NP,), jnp.float32),
        ],
        mesh=_SC_MESH,
        compiler_params=pltpu.CompilerParams(needs_layout_passes=False, use_tc_tiling_on_sc=False),
        scratch_types=[
            pltpu.VMEM((NP,), jnp.float32),
            pltpu.VMEM((NP,), jnp.float32),
            pltpu.VMEM((NCH, K), jnp.int32),
            pltpu.VMEM((NCH, K), jnp.int32),
            pltpu.VMEM((NCH, K), jnp.float32),
            pltpu.VMEM((K, c), jnp.float32),
            pltpu.VMEM((K, c), jnp.float32),
            pltpu.VMEM_SHARED((NP, c), jnp.float32),
            pltpu.VMEM_SHARED((NP,), jnp.float32),
            pltpu.SemaphoreType.DMA,
            pltpu.SemaphoreType.DMA,
        ],
    )


def _make_sc_pair_gather():
    c = 4

    def body(h_hbm, src_hbm, dst_hbm, hr_out, hc_out,
             h3_v, src_v, dst_v, rbuf, cbuf):
        cid = lax.axis_index("c")
        sid = lax.axis_index("s")
        wid = cid * 16 + sid
        pltpu.sync_copy(h_hbm, h3_v)
        pltpu.sync_copy(src_hbm.at[pl.ds(wid * NCH, NCH)], src_v)
        pltpu.sync_copy(dst_hbm.at[pl.ds(wid * NCH, NCH)], dst_v)

        lanes = _iota16()
        e4 = lanes >> 2
        f4 = lanes & 3

        def chunk(i, _):
            irow = jnp.full((16,), i, jnp.int32)

            def grp(g, _):
                eloc = g * 4 + e4
                nr = plsc.load_gather(src_v, [irow, eloc])
                nc = plsc.load_gather(dst_v, [irow, eloc])
                plsc.store_scatter(rbuf, [eloc, f4],
                                   plsc.load_gather(h3_v, [nr, f4]))
                plsc.store_scatter(cbuf, [eloc, f4],
                                   plsc.load_gather(h3_v, [nc, f4]))
                return 0

            lax.fori_loop(0, K // 4, grp, 0)
            base = wid * PW + i * K
            pltpu.sync_copy(rbuf, hr_out.at[pl.ds(base, K)])
            pltpu.sync_copy(cbuf, hc_out.at[pl.ds(base, K)])
            return 0

        lax.fori_loop(0, NCH, chunk, 0)

    return pl.kernel(
        body,
        out_type=[
            jax.ShapeDtypeStruct((EP, c), jnp.float32),
            jax.ShapeDtypeStruct((EP, c), jnp.float32),
        ],
        mesh=_SC_MESH,
        compiler_params=pltpu.CompilerParams(needs_layout_passes=False, use_tc_tiling_on_sc=False),
        scratch_types=[
            pltpu.VMEM((NP, c), jnp.float32),
            pltpu.VMEM((NCH, K), jnp.int32),
            pltpu.VMEM((NCH, K), jnp.int32),
            pltpu.VMEM((K, c), jnp.float32),
            pltpu.VMEM((K, c), jnp.float32),
        ],
    )


_sc_agg_128 = _make_sc_aggregate(128)
_sc_agg_64 = _make_sc_aggregate(64)
_sc_agg_16 = _make_sc_aggregate_es(16)
_sc_pair = _make_sc_pair_gather()


def _mlp_body(hr_ref, hc_ref, ea_ref, w1r_ref, w1c_ref, w1e_ref, b1_ref,
              w2_ref, b2_ref, w3_ref, b3_ref, out_ref):
    z = (hr_ref[...] @ w1r_ref[...] + hc_ref[...] @ w1c_ref[...]
         + ea_ref[...] @ w1e_ref[...] + b1_ref[...])
    z = jnp.maximum(z, 0.0)
    z = jnp.maximum(z @ w2_ref[...] + b2_ref[...], 0.0)
    z = z @ w3_ref[...] + b3_ref[...]
    out_ref[...] = ea_ref[...] + z


def _edge_mlp(hr, hc, ea, Wm1, bm1, Wm2, bm2, Wm3, bm3):
    E = ea.shape[0]
    hid = Wm1.shape[1]
    edim = Wm3.shape[1]
    grid = (E // MLP_BLK,)
    return pl.pallas_call(
        _mlp_body,
        grid=grid,
        in_specs=[
            pl.BlockSpec((MLP_BLK, 4), lambda i: (i, 0)),
            pl.BlockSpec((MLP_BLK, 4), lambda i: (i, 0)),
            pl.BlockSpec((MLP_BLK, 16), lambda i: (i, 0)),
            pl.BlockSpec((4, hid), lambda i: (0, 0)),
            pl.BlockSpec((4, hid), lambda i: (0, 0)),
            pl.BlockSpec((16, hid), lambda i: (0, 0)),
            pl.BlockSpec((1, hid), lambda i: (0, 0)),
            pl.BlockSpec((hid, hid), lambda i: (0, 0)),
            pl.BlockSpec((1, hid), lambda i: (0, 0)),
            pl.BlockSpec((hid, edim), lambda i: (0, 0)),
            pl.BlockSpec((1, edim), lambda i: (0, 0)),
        ],
        out_specs=pl.BlockSpec((MLP_BLK, edim), lambda i: (i, 0)),
        out_shape=jax.ShapeDtypeStruct((E, edim), jnp.float32),
    )(hr, hc, ea, Wm1[:4], Wm1[4:8], Wm1[8:], bm1.reshape(1, -1),
      Wm2, bm2.reshape(1, -1), Wm3, bm3.reshape(1, -1))


def kernel(x, edge_index, edge_attr, W1, a_src1, a_dst1, b1, W2, a_src2, a_dst2, b2, W3, a_src3, a_dst3, b3, Wm1, bm1, Wm2, bm2, Wm3, bm3):
    N = x.shape[0]
    npad = EP - E_EDGES
    pad_i = jnp.arange(npad, dtype=jnp.int32)
    src = jnp.concatenate([edge_index[0].astype(jnp.int32), pad_i % N])
    dst = jnp.concatenate([edge_index[1].astype(jnp.int32),
                           N + (pad_i % (NP - N))])
    src2d = src.reshape(EP // K, K)
    dst2d = dst.reshape(EP // K, K)
    xp = jnp.concatenate(
        [x, jnp.zeros((NP - N, x.shape[1]), jnp.float32)], axis=0)

    zn = jnp.zeros((NP,), jnp.float32)
    flat = lambda a: a.reshape(-1)
    cat = lambda a: jnp.concatenate(
        [a[:, :a.shape[1] // 2], a[:, a.shape[1] // 2:]], axis=0)

    h, ts, td, ws, sacc = _tc_stage(xp, W1, a_src1, a_dst1)
    acc_cat, den = _sc_agg_128(cat(h), flat(ts), flat(td), src2d, dst2d,
                               cat(sacc), flat(ws))
    h, ts, td, ws, sacc = _tc_combine_stage(acc_cat, den[:NP], b1,
                                            W2, a_src2, a_dst2)
    acc_cat, den = _sc_agg_64(cat(h), flat(ts), flat(td), src2d, dst2d,
                              cat(sacc), flat(ws))
    W3p = jnp.pad(W3, ((0, 0), (0, 12)))
    a_src3p = jnp.pad(a_src3, (0, 12))
    a_dst3p = jnp.pad(a_dst3, (0, 12))
    b3p = jnp.pad(b3, (0, 12))
    h, ts, td, ws, sacc = _tc_combine_stage(acc_cat, den[:NP], b2,
                                            W3p, a_src3p, a_dst3p)
    acc_cat, den_cat = _sc_agg_16(
        h, flat(ts), flat(td), src2d, dst2d,
        jnp.concatenate([sacc, jnp.zeros((NP, 16), jnp.float32)], axis=0),
        jnp.concatenate([flat(ws), zn]))
    h3 = _tc_combine(acc_cat, den_cat, b3p)[:, :4]
    hr, hc = _sc_pair(h3, src2d, dst2d)
    return _edge_mlp(hr[:E_EDGES], hc[:E_EDGES], edge_attr,
                     Wm1, bm1, Wm2, bm2, Wm3, bm3)


# speedup vs baseline: 15.8402x; 1.0619x over previous
import functools

import jax
import jax.numpy as jnp
from jax import lax
from jax.experimental import pallas as pl
from jax.experimental.pallas import tpu as pltpu
from jax.experimental.pallas import tpu_sc as plsc


N_NODES = 10000
NP = 10240
E_EDGES = 160000
EP = 163840
PW = EP // 32
K = 128
NCH = PW // K
RPS = NP // 16
MLP_BLK = 3200
TC_BLK = 1280

_SC_MESH = plsc.VectorSubcoreMesh(core_axis_name="c", subcore_axis_name="s",
                                  num_cores=2, num_subcores=16)


def _tc_stage_body(x_ref, w_ref, asrc_ref, adst_ref,
                   h_ref, ts_ref, td_ref, wself_ref, sacc_ref):
    h = x_ref[...] @ w_ref[...]
    ts = h @ asrc_ref[...]
    td = h @ adst_ref[...]
    e = ts + td
    ws = jnp.exp(jnp.maximum(e, 0.2 * e))
    h_ref[...] = h
    ts_ref[...] = ts
    td_ref[...] = td
    wself_ref[...] = ws
    sacc_ref[...] = h * ws


def _tc_stage(x, W, a_src, a_dst):
    cin, c = W.shape
    grid = (NP // TC_BLK,)
    return pl.pallas_call(
        _tc_stage_body,
        grid=grid,
        in_specs=[
            pl.BlockSpec((TC_BLK, cin), lambda i: (i, 0)),
            pl.BlockSpec((cin, c), lambda i: (0, 0)),
            pl.BlockSpec((c, 1), lambda i: (0, 0)),
            pl.BlockSpec((c, 1), lambda i: (0, 0)),
        ],
        out_specs=[
            pl.BlockSpec((TC_BLK, c), lambda i: (i, 0)),
            pl.BlockSpec((TC_BLK, 1), lambda i: (i, 0)),
            pl.BlockSpec((TC_BLK, 1), lambda i: (i, 0)),
            pl.BlockSpec((TC_BLK, 1), lambda i: (i, 0)),
            pl.BlockSpec((TC_BLK, c), lambda i: (i, 0)),
        ],
        out_shape=[
            jax.ShapeDtypeStruct((NP, c), jnp.float32),
            jax.ShapeDtypeStruct((NP, 1), jnp.float32),
            jax.ShapeDtypeStruct((NP, 1), jnp.float32),
            jax.ShapeDtypeStruct((NP, 1), jnp.float32),
            jax.ShapeDtypeStruct((NP, c), jnp.float32),
        ],
    )(x, W, a_src.reshape(-1, 1), a_dst.reshape(-1, 1))


def _tc_combine_stage_body(al_ref, ar_ref, d_ref, b_ref,
                           w_ref, asrc_ref, adst_ref,
                           h_ref, ts_ref, td_ref, wself_ref, sacc_ref):
    acc = jnp.concatenate([al_ref[...], ar_ref[...]], axis=-1)
    x = acc / (d_ref[...] + 1e-16) + b_ref[...]
    h = x @ w_ref[...]
    ts = h @ asrc_ref[...]
    td = h @ adst_ref[...]
    e = ts + td
    ws = jnp.exp(jnp.maximum(e, 0.2 * e))
    h_ref[...] = h
    ts_ref[...] = ts
    td_ref[...] = td
    wself_ref[...] = ws
    sacc_ref[...] = h * ws


def _tc_combine_stage(acc_cat, den, b, W, a_src, a_dst):
    cin, c = W.shape
    hcin = cin // 2
    nb = NP // TC_BLK
    grid = (nb,)
    return pl.pallas_call(
        _tc_combine_stage_body,
        grid=grid,
        in_specs=[
            pl.BlockSpec((TC_BLK, hcin), lambda i: (i, 0)),
            pl.BlockSpec((TC_BLK, hcin), lambda i: (i + NP // TC_BLK, 0)),
            pl.BlockSpec((TC_BLK, 1), lambda i: (i, 0)),
            pl.BlockSpec((1, cin), lambda i: (0, 0)),
            pl.BlockSpec((cin, c), lambda i: (0, 0)),
            pl.BlockSpec((c, 1), lambda i: (0, 0)),
            pl.BlockSpec((c, 1), lambda i: (0, 0)),
        ],
        out_specs=[
            pl.BlockSpec((TC_BLK, c), lambda i: (i, 0)),
            pl.BlockSpec((TC_BLK, 1), lambda i: (i, 0)),
            pl.BlockSpec((TC_BLK, 1), lambda i: (i, 0)),
            pl.BlockSpec((TC_BLK, 1), lambda i: (i, 0)),
            pl.BlockSpec((TC_BLK, c), lambda i: (i, 0)),
        ],
        out_shape=[
            jax.ShapeDtypeStruct((NP, c), jnp.float32),
            jax.ShapeDtypeStruct((NP, 1), jnp.float32),
            jax.ShapeDtypeStruct((NP, 1), jnp.float32),
            jax.ShapeDtypeStruct((NP, 1), jnp.float32),
            jax.ShapeDtypeStruct((NP, c), jnp.float32),
        ],
    )(acc_cat, acc_cat, den.reshape(-1, 1), b.reshape(1, -1), W,
      a_src.reshape(-1, 1), a_dst.reshape(-1, 1))


def _tc_combine_body(a0_ref, a1_ref, d0_ref, d1_ref, b_ref, out_ref):
    out_ref[...] = ((a0_ref[...] + a1_ref[...])
                    / (d0_ref[...] + d1_ref[...] + 1e-16) + b_ref[...])


def _tc_combine(acc_cat, den_cat, b):
    c = acc_cat.shape[1]
    grid = (NP // TC_BLK,)
    return pl.pallas_call(
        _tc_combine_body,
        grid=grid,
        in_specs=[
            pl.BlockSpec((TC_BLK, c), lambda i: (i, 0)),
            pl.BlockSpec((TC_BLK, c), lambda i: (i + NP // TC_BLK, 0)),
            pl.BlockSpec((TC_BLK, 1), lambda i: (i, 0)),
            pl.BlockSpec((TC_BLK, 1), lambda i: (i + NP // TC_BLK, 0)),
            pl.BlockSpec((1, c), lambda i: (0, 0)),
        ],
        out_specs=pl.BlockSpec((TC_BLK, c), lambda i: (i, 0)),
        out_shape=jax.ShapeDtypeStruct((NP, c), jnp.float32),
    )(acc_cat, acc_cat, den_cat.reshape(-1, 1), den_cat.reshape(-1, 1),
      b.reshape(1, -1))


def _iota16():
    return lax.iota(jnp.int32, 16)


NCH2 = EP // 16 // K


def _make_sc_aggregate(c):
    hc = c // 2

    def body(hcat_hbm, ts_hbm, td_hbm, src_hbm, dst_hbm,
             init_hbm, wself_hbm,
             acc_out, den_out,
             ts_v, td_v, src_v, dst_v, w_v, rows0, rows1,
             acc_sh, den_sh, sem0, sem1):
        cid = lax.axis_index("c")
        sid = lax.axis_index("s")
        r0 = sid * RPS
        pltpu.sync_copy(init_hbm.at[pl.ds(cid * NP + r0, RPS)],
                        acc_sh.at[pl.ds(r0, RPS)])
        pltpu.sync_copy(wself_hbm.at[pl.ds(r0, RPS)],
                        den_sh.at[pl.ds(r0, RPS)])

        pltpu.sync_copy(ts_hbm, ts_v)
        pltpu.sync_copy(td_hbm, td_v)
        pltpu.sync_copy(src_hbm.at[pl.ds(sid * NCH2, NCH2)], src_v)
        pltpu.sync_copy(dst_hbm.at[pl.ds(sid * NCH2, NCH2)], dst_v)

        off = jnp.full((16,), cid * NP, jnp.int32)

        def wchunk(i, _):
            for j in range(K // 16):
                sl = pl.ds(j * 16, 16)
                sidx = src_v[i, sl]
                didx = dst_v[i, sl]
                e = (plsc.load_gather(ts_v, [sidx])
                     + plsc.load_gather(td_v, [didx]))
                w_v[i, sl] = jnp.exp(jnp.maximum(e, 0.2 * e))
                src_v[i, sl] = sidx + off
            return 0

        lax.fori_loop(0, NCH2, wchunk, 0)
        plsc.subcore_barrier()

        pltpu.async_copy(hcat_hbm.at[src_v.at[0]], rows0, sem0)

        def chunk(i, cur, csem, pf, nxt, nsem):
            pltpu.make_async_copy(hcat_hbm.at[src_v.at[i]], cur, csem).wait()

            @pl.when(pf < NCH2)
            def _():
                pltpu.async_copy(hcat_hbm.at[src_v.at[pf]], nxt, nsem)

            @plsc.parallel_loop(0, K, unroll=8)
            def mul_row(e2):
                wb = plsc.load_gather(
                    w_v, [jnp.full((16,), i, jnp.int32),
                          jnp.full((16,), e2, jnp.int32)])
                for jj in range(hc // 16):
                    sl2 = pl.ds(jj * 16, 16)
                    cur[e2, sl2] = cur[e2, sl2] * wb
            pltpu.sync_copy(cur, acc_sh.at[dst_v.at[i]], add=True)
            pltpu.sync_copy(w_v.at[i], den_sh.at[dst_v.at[i]], add=True)

        def pair(p, _):
            i0 = p * 2
            chunk(i0, rows0, sem0, i0 + 1, rows1, sem1)
            chunk(i0 + 1, rows1, sem1, i0 + 2, rows0, sem0)
            return 0

        lax.fori_loop(0, NCH2 // 2, pair, 0)
        plsc.subcore_barrier()
        pltpu.sync_copy(acc_sh.at[pl.ds(r0, RPS)],
                        acc_out.at[pl.ds(cid * NP + r0, RPS)])
        pltpu.sync_copy(den_sh.at[pl.ds(r0, RPS)],
                        den_out.at[pl.ds(cid * NP + r0, RPS)])

    return pl.kernel(
        body,
        out_type=[
            jax.ShapeDtypeStruct((2 * NP, hc), jnp.float32),
            jax.ShapeDtypeStruct((2 * NP,), jnp.float32),
        ],
        mesh=_SC_MESH,
        compiler_params=pltpu.CompilerParams(needs_layout_passes=False, use_tc_tiling_on_sc=False),
        scratch_types=[
            pltpu.VMEM((NP,), jnp.float32),
            pltpu.VMEM((NP,), jnp.float32),
            pltpu.VMEM((NCH2, K), jnp.int32),
            pltpu.VMEM((NCH2, K), jnp.int32),
            pltpu.VMEM((NCH2, K), jnp.float32),
            pltpu.VMEM((K, hc), jnp.float32),
            pltpu.VMEM((K, hc), jnp.float32),
            pltpu.VMEM_SHARED((NP, hc), jnp.float32),
            pltpu.VMEM_SHARED((NP,), jnp.float32),
            pltpu.SemaphoreType.DMA,
            pltpu.SemaphoreType.DMA,
        ],
    )


def _make_sc_aggregate_es(c):

    def body(h_hbm, ts_hbm, td_hbm, src_hbm, dst_hbm,
             initcat_hbm, wscat_hbm,
             acc_out, den_out,
             ts_v, td_v, src_v, dst_v, w_v, rows0, rows1,
             acc_sh, den_sh, sem0, sem1):
        cid = lax.axis_index("c")
        sid = lax.axis_index("s")
        wid = cid * 16 + sid
        r0 = sid * RPS
        pltpu.sync_copy(initcat_hbm.at[pl.ds(cid * NP + r0, RPS)],
                        acc_sh.at[pl.ds(r0, RPS)])
        pltpu.sync_copy(wscat_hbm.at[pl.ds(cid * NP + r0, RPS)],
                        den_sh.at[pl.ds(r0, RPS)])

        pltpu.sync_copy(ts_hbm, ts_v)
        pltpu.sync_copy(td_hbm, td_v)
        pltpu.sync_copy(src_hbm.at[pl.ds(wid * NCH, NCH)], src_v)
        pltpu.sync_copy(dst_hbm.at[pl.ds(wid * NCH, NCH)], dst_v)

        def wchunk(i, _):
            for j in range(K // 16):
                sl = pl.ds(j * 16, 16)
                sidx = src_v[i, sl]
                didx = dst_v[i, sl]
                e = (plsc.load_gather(ts_v, [sidx])
                     + plsc.load_gather(td_v, [didx]))
                w_v[i, sl] = jnp.exp(jnp.maximum(e, 0.2 * e))
            return 0

        lax.fori_loop(0, NCH, wchunk, 0)
        plsc.subcore_barrier()

        pltpu.async_copy(h_hbm.at[src_v.at[0]], rows0, sem0)

        def chunk(i, cur, csem, pf, nxt, nsem):
            pltpu.make_async_copy(h_hbm.at[src_v.at[i]], cur, csem).wait()

            @pl.when(pf < NCH)
            def _():
                pltpu.async_copy(h_hbm.at[src_v.at[pf]], nxt, nsem)

            @plsc.parallel_loop(0, K, unroll=8)
            def mul_row(e2):
                wb = plsc.load_gather(
                    w_v, [jnp.full((16,), i, jnp.int32),
                          jnp.full((16,), e2, jnp.int32)])
                for jj in range(c // 16):
                    sl2 = pl.ds(jj * 16, 16)
                    cur[e2, sl2] = cur[e2, sl2] * wb
            pltpu.sync_copy(cur, acc_sh.at[dst_v.at[i]], add=True)
            pltpu.sync_copy(w_v.at[i], den_sh.at[dst_v.at[i]], add=True)

        def pair(p, _):
            i0 = p * 2
            chunk(i0, rows0, sem0, i0 + 1, rows1, sem1)
            chunk(i0 + 1, rows1, sem1, i0 + 2, rows0, sem0)
            return 0

        lax.fori_loop(0, NCH // 2, pair, 0)
        plsc.subcore_barrier()
        pltpu.sync_copy(acc_sh.at[pl.ds(r0, RPS)],
                        acc_out.at[pl.ds(cid * NP + r0, RPS)])
        pltpu.sync_copy(den_sh.at[pl.ds(r0, RPS)],
                        den_out.at[pl.ds(cid * NP + r0, RPS)])

    return pl.kernel(
        body,
        out_type=[
            jax.ShapeDtypeStruct((2 * NP, c), jnp.float32),
            jax.ShapeDtypeStruct((2 * NP,), jnp.float32),
        ],
        mesh=_SC_MESH,
        compiler_params=pltpu.CompilerParams(needs_layout_passes=False, use_tc_tiling_on_sc=False),
        scratch_types=[
            pltpu.VMEM((NP,), jnp.float32),
            pltpu.VMEM((NP,), jnp.float32),
            pltpu.VMEM((NCH, K), jnp.int32),
            pltpu.VMEM((NCH, K), jnp.int32),
            pltpu.VMEM((NCH, K), jnp.float32),
            pltpu.VMEM((K, c), jnp.float32),
            pltpu.VMEM((K, c), jnp.float32),
            pltpu.VMEM_SHARED((NP, c), jnp.float32),
            pltpu.VMEM_SHARED((NP,), jnp.float32),
            pltpu.SemaphoreType.DMA,
            pltpu.SemaphoreType.DMA,
        ],
    )


def _make_sc_pair_gather():
    c = 16

    def body(h_hbm, src_hbm, dst_hbm, hr_out, hc_out,
             src_v, dst_v, rbuf, cbuf, sem0, sem1):
        cid = lax.axis_index("c")
        sid = lax.axis_index("s")
        wid = cid * 16 + sid
        pltpu.sync_copy(src_hbm.at[pl.ds(wid * NCH, NCH)], src_v)
        pltpu.sync_copy(dst_hbm.at[pl.ds(wid * NCH, NCH)], dst_v)

        def chunk(i, _):
            pltpu.async_copy(h_hbm.at[src_v.at[i]], rbuf, sem0)
            pltpu.async_copy(h_hbm.at[dst_v.at[i]], cbuf, sem1)
            pltpu.make_async_copy(h_hbm.at[src_v.at[i]], rbuf, sem0).wait()
            pltpu.make_async_copy(h_hbm.at[dst_v.at[i]], cbuf, sem1).wait()
            base = wid * PW + i * K
            pltpu.sync_copy(rbuf, hr_out.at[pl.ds(base, K)])
            pltpu.sync_copy(cbuf, hc_out.at[pl.ds(base, K)])
            return 0

        lax.fori_loop(0, NCH, chunk, 0)

    return pl.kernel(
        body,
        out_type=[
            jax.ShapeDtypeStruct((EP, c), jnp.float32),
            jax.ShapeDtypeStruct((EP, c), jnp.float32),
        ],
        mesh=_SC_MESH,
        compiler_params=pltpu.CompilerParams(needs_layout_passes=False, use_tc_tiling_on_sc=False),
        scratch_types=[
            pltpu.VMEM((NCH, K), jnp.int32),
            pltpu.VMEM((NCH, K), jnp.int32),
            pltpu.VMEM((K, c), jnp.float32),
            pltpu.VMEM((K, c), jnp.float32),
            pltpu.SemaphoreType.DMA,
            pltpu.SemaphoreType.DMA,
        ],
    )


_sc_agg_128 = _make_sc_aggregate(128)
_sc_agg_64 = _make_sc_aggregate(64)
_sc_agg_16 = _make_sc_aggregate_es(16)
_sc_pair = _make_sc_pair_gather()


def _mlp_body(hr_ref, hc_ref, ea_ref, w1r_ref, w1c_ref, w1e_ref, b1_ref,
              w2_ref, b2_ref, w3_ref, b3_ref, out_ref):
    z = (hr_ref[...] @ w1r_ref[...] + hc_ref[...] @ w1c_ref[...]
         + ea_ref[...] @ w1e_ref[...] + b1_ref[...])
    z = jnp.maximum(z, 0.0)
    z = jnp.dot(z.astype(jnp.bfloat16), w2_ref[...],
                preferred_element_type=jnp.float32) + b2_ref[...]
    z = jnp.maximum(z, 0.0)
    z = jnp.dot(z.astype(jnp.bfloat16), w3_ref[...],
                preferred_element_type=jnp.float32) + b3_ref[...]
    out_ref[...] = ea_ref[...] + z


def _edge_mlp(hr, hc, ea, Wm1, bm1, Wm2, bm2, Wm3, bm3):
    E = ea.shape[0]
    hid = Wm1.shape[1]
    edim = Wm3.shape[1]
    grid = (E // MLP_BLK,)
    return pl.pallas_call(
        _mlp_body,
        grid=grid,
        in_specs=[
            pl.BlockSpec((MLP_BLK, 16), lambda i: (i, 0)),
            pl.BlockSpec((MLP_BLK, 16), lambda i: (i, 0)),
            pl.BlockSpec((MLP_BLK, 16), lambda i: (i, 0)),
            pl.BlockSpec((16, hid), lambda i: (0, 0)),
            pl.BlockSpec((16, hid), lambda i: (0, 0)),
            pl.BlockSpec((16, hid), lambda i: (0, 0)),
            pl.BlockSpec((1, hid), lambda i: (0, 0)),
            pl.BlockSpec((hid, hid), lambda i: (0, 0)),
            pl.BlockSpec((1, hid), lambda i: (0, 0)),
            pl.BlockSpec((hid, edim), lambda i: (0, 0)),
            pl.BlockSpec((1, edim), lambda i: (0, 0)),
        ],
        out_specs=pl.BlockSpec((MLP_BLK, edim), lambda i: (i, 0)),
        out_shape=jax.ShapeDtypeStruct((E, edim), jnp.float32),
    )(hr, hc, ea,
      jnp.pad(Wm1[:4], ((0, 12), (0, 0))), jnp.pad(Wm1[4:8], ((0, 12), (0, 0))),
      Wm1[8:], bm1.reshape(1, -1),
      Wm2.astype(jnp.bfloat16), bm2.reshape(1, -1),
      Wm3.astype(jnp.bfloat16), bm3.reshape(1, -1))


def kernel(x, edge_index, edge_attr, W1, a_src1, a_dst1, b1, W2, a_src2, a_dst2, b2, W3, a_src3, a_dst3, b3, Wm1, bm1, Wm2, bm2, Wm3, bm3):
    N = x.shape[0]
    npad = EP - E_EDGES
    pad_i = jnp.arange(npad, dtype=jnp.int32)
    src = jnp.concatenate([edge_index[0].astype(jnp.int32), pad_i % N])
    dst = jnp.concatenate([edge_index[1].astype(jnp.int32),
                           N + (pad_i % (NP - N))])
    src2d = src.reshape(EP // K, K)
    dst2d = dst.reshape(EP // K, K)
    xp = jnp.concatenate(
        [x, jnp.zeros((NP - N, x.shape[1]), jnp.float32)], axis=0)

    zn = jnp.zeros((NP,), jnp.float32)
    flat = lambda a: a.reshape(-1)
    cat = lambda a: jnp.concatenate(
        [a[:, :a.shape[1] // 2], a[:, a.shape[1] // 2:]], axis=0)

    h, ts, td, ws, sacc = _tc_stage(xp, W1, a_src1, a_dst1)
    acc_cat, den = _sc_agg_128(cat(h), flat(ts), flat(td), src2d, dst2d,
                               cat(sacc), flat(ws))
    h, ts, td, ws, sacc = _tc_combine_stage(acc_cat, den[:NP], b1,
                                            W2, a_src2, a_dst2)
    acc_cat, den = _sc_agg_64(cat(h), flat(ts), flat(td), src2d, dst2d,
                              cat(sacc), flat(ws))
    W3p = jnp.pad(W3, ((0, 0), (0, 12)))
    a_src3p = jnp.pad(a_src3, (0, 12))
    a_dst3p = jnp.pad(a_dst3, (0, 12))
    b3p = jnp.pad(b3, (0, 12))
    h, ts, td, ws, sacc = _tc_combine_stage(acc_cat, den[:NP], b2,
                                            W3p, a_src3p, a_dst3p)
    acc_cat, den_cat = _sc_agg_16(
        h, flat(ts), flat(td), src2d, dst2d,
        jnp.concatenate([sacc, jnp.zeros((NP, 16), jnp.float32)], axis=0),
        jnp.concatenate([flat(ws), zn]))
    h3 = _tc_combine(acc_cat, den_cat, b3p)
    hr, hc = _sc_pair(h3, src2d, dst2d)
    return _edge_mlp(hr[:E_EDGES], hc[:E_EDGES], edge_attr,
                     Wm1, bm1, Wm2, bm2, Wm3, bm3)


# speedup vs baseline: 19.5831x; 1.2363x over previous
import functools

import jax
import jax.numpy as jnp
from jax import lax
from jax.experimental import pallas as pl
from jax.experimental.pallas import tpu as pltpu
from jax.experimental.pallas import tpu_sc as plsc


N_NODES = 10000
NP = 10240
E_EDGES = 160000
EP = 163840
PW = EP // 32
K = 128
NCH = PW // K
RPS = NP // 16
MLP_BLK = 3200
TC_BLK = 1280

_SC_MESH = plsc.VectorSubcoreMesh(core_axis_name="c", subcore_axis_name="s",
                                  num_cores=2, num_subcores=16)


def _tc_stage_body(x_ref, w_ref, asrc_ref, adst_ref,
                   h_ref, ts_ref, td_ref, wself_ref, sacc_ref):
    h = x_ref[...] @ w_ref[...]
    ts = h @ asrc_ref[...]
    td = h @ adst_ref[...]
    e = ts + td
    ws = jnp.exp(jnp.maximum(e, 0.2 * e))
    h_ref[...] = h
    ts_ref[...] = ts
    td_ref[...] = td
    wself_ref[...] = ws
    sacc_ref[...] = h * ws


def _tc_stage(x, W, a_src, a_dst):
    cin, c = W.shape
    grid = (NP // TC_BLK,)
    return pl.pallas_call(
        _tc_stage_body,
        grid=grid,
        in_specs=[
            pl.BlockSpec((TC_BLK, cin), lambda i: (i, 0)),
            pl.BlockSpec((cin, c), lambda i: (0, 0)),
            pl.BlockSpec((c, 1), lambda i: (0, 0)),
            pl.BlockSpec((c, 1), lambda i: (0, 0)),
        ],
        out_specs=[
            pl.BlockSpec((TC_BLK, c), lambda i: (i, 0)),
            pl.BlockSpec((TC_BLK, 1), lambda i: (i, 0)),
            pl.BlockSpec((TC_BLK, 1), lambda i: (i, 0)),
            pl.BlockSpec((TC_BLK, 1), lambda i: (i, 0)),
            pl.BlockSpec((TC_BLK, c), lambda i: (i, 0)),
        ],
        out_shape=[
            jax.ShapeDtypeStruct((NP, c), jnp.float32),
            jax.ShapeDtypeStruct((NP, 1), jnp.float32),
            jax.ShapeDtypeStruct((NP, 1), jnp.float32),
            jax.ShapeDtypeStruct((NP, 1), jnp.float32),
            jax.ShapeDtypeStruct((NP, c), jnp.float32),
        ],
    )(x, W, a_src.reshape(-1, 1), a_dst.reshape(-1, 1))


def _tc_combine_stage_body(al_ref, ar_ref, d_ref, b_ref,
                           w_ref, asrc_ref, adst_ref,
                           h_ref, ts_ref, td_ref, wself_ref, sacc_ref):
    acc = jnp.concatenate([al_ref[...], ar_ref[...]], axis=-1)
    x = acc / (d_ref[...] + 1e-16) + b_ref[...]
    h = x @ w_ref[...]
    ts = h @ asrc_ref[...]
    td = h @ adst_ref[...]
    e = ts + td
    ws = jnp.exp(jnp.maximum(e, 0.2 * e))
    h_ref[...] = h
    ts_ref[...] = ts
    td_ref[...] = td
    wself_ref[...] = ws
    sacc_ref[...] = h * ws


def _tc_combine_stage(acc_cat, den, b, W, a_src, a_dst):
    cin, c = W.shape
    hcin = cin // 2
    nb = NP // TC_BLK
    grid = (nb,)
    return pl.pallas_call(
        _tc_combine_stage_body,
        grid=grid,
        in_specs=[
            pl.BlockSpec((TC_BLK, hcin), lambda i: (i, 0)),
            pl.BlockSpec((TC_BLK, hcin), lambda i: (i + NP // TC_BLK, 0)),
            pl.BlockSpec((TC_BLK, 1), lambda i: (i, 0)),
            pl.BlockSpec((1, cin), lambda i: (0, 0)),
            pl.BlockSpec((cin, c), lambda i: (0, 0)),
            pl.BlockSpec((c, 1), lambda i: (0, 0)),
            pl.BlockSpec((c, 1), lambda i: (0, 0)),
        ],
        out_specs=[
            pl.BlockSpec((TC_BLK, c), lambda i: (i, 0)),
            pl.BlockSpec((TC_BLK, 1), lambda i: (i, 0)),
            pl.BlockSpec((TC_BLK, 1), lambda i: (i, 0)),
            pl.BlockSpec((TC_BLK, 1), lambda i: (i, 0)),
            pl.BlockSpec((TC_BLK, c), lambda i: (i, 0)),
        ],
        out_shape=[
            jax.ShapeDtypeStruct((NP, c), jnp.float32),
            jax.ShapeDtypeStruct((NP, 1), jnp.float32),
            jax.ShapeDtypeStruct((NP, 1), jnp.float32),
            jax.ShapeDtypeStruct((NP, 1), jnp.float32),
            jax.ShapeDtypeStruct((NP, c), jnp.float32),
        ],
    )(acc_cat, acc_cat, den.reshape(-1, 1), b.reshape(1, -1), W,
      a_src.reshape(-1, 1), a_dst.reshape(-1, 1))


def _tc_combine_body(a0_ref, a1_ref, d0_ref, d1_ref, b_ref, out_ref):
    out_ref[...] = ((a0_ref[...] + a1_ref[...])
                    / (d0_ref[...] + d1_ref[...] + 1e-16) + b_ref[...])


def _tc_combine(acc_cat, den_cat, b):
    c = acc_cat.shape[1]
    grid = (NP // TC_BLK,)
    return pl.pallas_call(
        _tc_combine_body,
        grid=grid,
        in_specs=[
            pl.BlockSpec((TC_BLK, c), lambda i: (i, 0)),
            pl.BlockSpec((TC_BLK, c), lambda i: (i + NP // TC_BLK, 0)),
            pl.BlockSpec((TC_BLK, 1), lambda i: (i, 0)),
            pl.BlockSpec((TC_BLK, 1), lambda i: (i + NP // TC_BLK, 0)),
            pl.BlockSpec((1, c), lambda i: (0, 0)),
        ],
        out_specs=pl.BlockSpec((TC_BLK, c), lambda i: (i, 0)),
        out_shape=jax.ShapeDtypeStruct((NP, c), jnp.float32),
    )(acc_cat, acc_cat, den_cat.reshape(-1, 1), den_cat.reshape(-1, 1),
      b.reshape(1, -1))


def _iota16():
    return lax.iota(jnp.int32, 16)


NCH2 = EP // 16 // K


def _make_sc_aggregate(c):
    hc = c // 2

    def body(hcat_hbm, ts_hbm, td_hbm, src_hbm, dst_hbm,
             init_hbm, wself_hbm,
             acc_out, den_out,
             ts_v, td_v, src_v, dst_v, w_v, rows0, rows1,
             acc_sh, den_sh, sem0, sem1):
        cid = lax.axis_index("c")
        sid = lax.axis_index("s")
        r0 = sid * RPS
        pltpu.sync_copy(init_hbm.at[pl.ds(cid * NP + r0, RPS)],
                        acc_sh.at[pl.ds(r0, RPS)])
        pltpu.sync_copy(wself_hbm.at[pl.ds(r0, RPS)],
                        den_sh.at[pl.ds(r0, RPS)])

        pltpu.sync_copy(ts_hbm, ts_v)
        pltpu.sync_copy(td_hbm, td_v)
        pltpu.sync_copy(src_hbm.at[pl.ds(sid * NCH2, NCH2)], src_v)
        pltpu.sync_copy(dst_hbm.at[pl.ds(sid * NCH2, NCH2)], dst_v)

        off = jnp.full((16,), cid * NP, jnp.int32)

        def wchunk(i, _):
            for j in range(K // 16):
                sl = pl.ds(j * 16, 16)
                sidx = src_v[i, sl]
                didx = dst_v[i, sl]
                e = (plsc.load_gather(ts_v, [sidx])
                     + plsc.load_gather(td_v, [didx]))
                w_v[i, sl] = jnp.exp(jnp.maximum(e, 0.2 * e))
                src_v[i, sl] = sidx + off
            return 0

        lax.fori_loop(0, NCH2, wchunk, 0)
        plsc.subcore_barrier()

        pltpu.async_copy(hcat_hbm.at[src_v.at[0]], rows0, sem0)

        def chunk(i, cur, csem, pf, nxt, nsem):
            pltpu.make_async_copy(hcat_hbm.at[src_v.at[i]], cur, csem).wait()

            @pl.when(pf < NCH2)
            def _():
                pltpu.async_copy(hcat_hbm.at[src_v.at[pf]], nxt, nsem)

            @plsc.parallel_loop(0, K, unroll=8)
            def mul_row(e2):
                wb = plsc.load_gather(
                    w_v, [jnp.full((16,), i, jnp.int32),
                          jnp.full((16,), e2, jnp.int32)])
                for jj in range(hc // 16):
                    sl2 = pl.ds(jj * 16, 16)
                    cur[e2, sl2] = cur[e2, sl2] * wb
            pltpu.sync_copy(cur, acc_sh.at[dst_v.at[i]], add=True)
            pltpu.sync_copy(w_v.at[i], den_sh.at[dst_v.at[i]], add=True)

        def pair(p, _):
            i0 = p * 2
            chunk(i0, rows0, sem0, i0 + 1, rows1, sem1)
            chunk(i0 + 1, rows1, sem1, i0 + 2, rows0, sem0)
            return 0

        lax.fori_loop(0, NCH2 // 2, pair, 0)
        plsc.subcore_barrier()
        pltpu.sync_copy(acc_sh.at[pl.ds(r0, RPS)],
                        acc_out.at[pl.ds(cid * NP + r0, RPS)])
        pltpu.sync_copy(den_sh.at[pl.ds(r0, RPS)],
                        den_out.at[pl.ds(cid * NP + r0, RPS)])

    return pl.kernel(
        body,
        out_type=[
            jax.ShapeDtypeStruct((2 * NP, hc), jnp.float32),
            jax.ShapeDtypeStruct((2 * NP,), jnp.float32),
        ],
        mesh=_SC_MESH,
        compiler_params=pltpu.CompilerParams(needs_layout_passes=False, use_tc_tiling_on_sc=False),
        scratch_types=[
            pltpu.VMEM((NP,), jnp.float32),
            pltpu.VMEM((NP,), jnp.float32),
            pltpu.VMEM((NCH2, K), jnp.int32),
            pltpu.VMEM((NCH2, K), jnp.int32),
            pltpu.VMEM((NCH2, K), jnp.float32),
            pltpu.VMEM((K, hc), jnp.float32),
            pltpu.VMEM((K, hc), jnp.float32),
            pltpu.VMEM_SHARED((NP, hc), jnp.float32),
            pltpu.VMEM_SHARED((NP,), jnp.float32),
            pltpu.SemaphoreType.DMA,
            pltpu.SemaphoreType.DMA,
        ],
    )


def _make_sc_aggregate_es(c):

    def body(h_hbm, ts_hbm, td_hbm, src_hbm, dst_hbm,
             initcat_hbm, wscat_hbm,
             acc_out, den_out,
             ts_v, td_v, src_v, dst_v, w_v, rows0, rows1,
             acc_sh, den_sh, sem0, sem1):
        cid = lax.axis_index("c")
        sid = lax.axis_index("s")
        wid = cid * 16 + sid
        r0 = sid * RPS
        pltpu.sync_copy(initcat_hbm.at[pl.ds(cid * NP + r0, RPS)],
                        acc_sh.at[pl.ds(r0, RPS)])
        pltpu.sync_copy(wscat_hbm.at[pl.ds(cid * NP + r0, RPS)],
                        den_sh.at[pl.ds(r0, RPS)])

        pltpu.sync_copy(ts_hbm, ts_v)
        pltpu.sync_copy(td_hbm, td_v)
        pltpu.sync_copy(src_hbm.at[pl.ds(wid * NCH, NCH)], src_v)
        pltpu.sync_copy(dst_hbm.at[pl.ds(wid * NCH, NCH)], dst_v)

        def wchunk(i, _):
            for j in range(K // 16):
                sl = pl.ds(j * 16, 16)
                sidx = src_v[i, sl]
                didx = dst_v[i, sl]
                e = (plsc.load_gather(ts_v, [sidx])
                     + plsc.load_gather(td_v, [didx]))
                w_v[i, sl] = jnp.exp(jnp.maximum(e, 0.2 * e))
            return 0

        lax.fori_loop(0, NCH, wchunk, 0)
        plsc.subcore_barrier()

        pltpu.async_copy(h_hbm.at[src_v.at[0]], rows0, sem0)

        def chunk(i, cur, csem, pf, nxt, nsem):
            pltpu.make_async_copy(h_hbm.at[src_v.at[i]], cur, csem).wait()

            @pl.when(pf < NCH)
            def _():
                pltpu.async_copy(h_hbm.at[src_v.at[pf]], nxt, nsem)

            @plsc.parallel_loop(0, K, unroll=8)
            def mul_row(e2):
                wb = plsc.load_gather(
                    w_v, [jnp.full((16,), i, jnp.int32),
                          jnp.full((16,), e2, jnp.int32)])
                for jj in range(c // 16):
                    sl2 = pl.ds(jj * 16, 16)
                    cur[e2, sl2] = cur[e2, sl2] * wb
            pltpu.sync_copy(cur, acc_sh.at[dst_v.at[i]], add=True)
            pltpu.sync_copy(w_v.at[i], den_sh.at[dst_v.at[i]], add=True)

        def pair(p, _):
            i0 = p * 2
            chunk(i0, rows0, sem0, i0 + 1, rows1, sem1)
            chunk(i0 + 1, rows1, sem1, i0 + 2, rows0, sem0)
            return 0

        lax.fori_loop(0, NCH // 2, pair, 0)
        plsc.subcore_barrier()
        pltpu.sync_copy(acc_sh.at[pl.ds(r0, RPS)],
                        acc_out.at[pl.ds(cid * NP + r0, RPS)])
        pltpu.sync_copy(den_sh.at[pl.ds(r0, RPS)],
                        den_out.at[pl.ds(cid * NP + r0, RPS)])

    return pl.kernel(
        body,
        out_type=[
            jax.ShapeDtypeStruct((2 * NP, c), jnp.float32),
            jax.ShapeDtypeStruct((2 * NP,), jnp.float32),
        ],
        mesh=_SC_MESH,
        compiler_params=pltpu.CompilerParams(needs_layout_passes=False, use_tc_tiling_on_sc=False),
        scratch_types=[
            pltpu.VMEM((NP,), jnp.float32),
            pltpu.VMEM((NP,), jnp.float32),
            pltpu.VMEM((NCH, K), jnp.int32),
            pltpu.VMEM((NCH, K), jnp.int32),
            pltpu.VMEM((NCH, K), jnp.float32),
            pltpu.VMEM((K, c), jnp.float32),
            pltpu.VMEM((K, c), jnp.float32),
            pltpu.VMEM_SHARED((NP, c), jnp.float32),
            pltpu.VMEM_SHARED((NP,), jnp.float32),
            pltpu.SemaphoreType.DMA,
            pltpu.SemaphoreType.DMA,
        ],
    )


def _make_sc_pair_gather():
    def body(h_hbm, src_hbm, dst_hbm, hrc_out,
             h3_v, src_v, dst_v, rbuf):
        cid = lax.axis_index("c")
        sid = lax.axis_index("s")
        wid = cid * 16 + sid
        pltpu.sync_copy(h_hbm, h3_v)
        pltpu.sync_copy(src_hbm.at[pl.ds(wid * NCH, NCH)], src_v)
        pltpu.sync_copy(dst_hbm.at[pl.ds(wid * NCH, NCH)], dst_v)

        lanes = _iota16()
        e2 = lanes >> 3
        sel = (lanes >> 2) & 1
        f4 = lanes & 3

        def chunk(i, _):
            irow = jnp.full((16,), i, jnp.int32)

            @plsc.parallel_loop(0, K // 2, unroll=4)
            def grp(g):
                eloc = g * 2 + e2
                nr = plsc.load_gather(src_v, [irow, eloc])
                nc = plsc.load_gather(dst_v, [irow, eloc])
                node = jnp.where(sel == 1, nc, nr)
                rbuf[pl.ds(g * 16, 16)] = plsc.load_gather(h3_v, [node, f4])

            base = (wid * PW + i * K) * 8
            pltpu.sync_copy(rbuf, hrc_out.at[pl.ds(base, K * 8)])
            return 0

        lax.fori_loop(0, NCH, chunk, 0)

    return pl.kernel(
        body,
        out_type=jax.ShapeDtypeStruct((EP * 8,), jnp.float32),
        mesh=_SC_MESH,
        compiler_params=pltpu.CompilerParams(needs_layout_passes=False, use_tc_tiling_on_sc=False),
        scratch_types=[
            pltpu.VMEM((NP, 4), jnp.float32),
            pltpu.VMEM((NCH, K), jnp.int32),
            pltpu.VMEM((NCH, K), jnp.int32),
            pltpu.VMEM((K * 8,), jnp.float32),
        ],
    )


_sc_agg_128 = _make_sc_aggregate(128)
_sc_agg_64 = _make_sc_aggregate(64)
_sc_agg_16 = _make_sc_aggregate_es(16)
_sc_pair = _make_sc_pair_gather()


def _mlp_body(hrc_ref, ea_ref, w1p_ref, w1e_ref, b1_ref,
              w2_ref, b2_ref, w3_ref, b3_ref, out_ref):
    z = (jnp.dot(hrc_ref[...].astype(jnp.bfloat16), w1p_ref[...],
                 preferred_element_type=jnp.float32)
         + jnp.dot(ea_ref[...].astype(jnp.bfloat16), w1e_ref[...],
                   preferred_element_type=jnp.float32) + b1_ref[...])
    z = jnp.maximum(z, 0.0)
    z = jnp.dot(z.astype(jnp.bfloat16), w2_ref[...],
                preferred_element_type=jnp.float32) + b2_ref[...]
    z = jnp.maximum(z, 0.0)
    z = jnp.dot(z.astype(jnp.bfloat16), w3_ref[...],
                preferred_element_type=jnp.float32) + b3_ref[...]
    out_ref[...] = ea_ref[...] + z


def _edge_mlp(hrc, ea, Wm1, bm1, Wm2, bm2, Wm3, bm3):
    E = ea.shape[0]
    hid = Wm1.shape[1]
    edim = Wm3.shape[1]
    grid = (E // MLP_BLK,)
    return pl.pallas_call(
        _mlp_body,
        grid=grid,
        in_specs=[
            pl.BlockSpec((MLP_BLK, 8), lambda i: (i, 0)),
            pl.BlockSpec((MLP_BLK, 16), lambda i: (i, 0)),
            pl.BlockSpec((8, hid), lambda i: (0, 0)),
            pl.BlockSpec((16, hid), lambda i: (0, 0)),
            pl.BlockSpec((1, hid), lambda i: (0, 0)),
            pl.BlockSpec((hid, hid), lambda i: (0, 0)),
            pl.BlockSpec((1, hid), lambda i: (0, 0)),
            pl.BlockSpec((hid, edim), lambda i: (0, 0)),
            pl.BlockSpec((1, edim), lambda i: (0, 0)),
        ],
        out_specs=pl.BlockSpec((MLP_BLK, edim), lambda i: (i, 0)),
        out_shape=jax.ShapeDtypeStruct((E, edim), jnp.float32),
    )(hrc, ea,
      Wm1[:8].astype(jnp.bfloat16), Wm1[8:].astype(jnp.bfloat16),
      bm1.reshape(1, -1),
      Wm2.astype(jnp.bfloat16), bm2.reshape(1, -1),
      Wm3.astype(jnp.bfloat16), bm3.reshape(1, -1))


def kernel(x, edge_index, edge_attr, W1, a_src1, a_dst1, b1, W2, a_src2, a_dst2, b2, W3, a_src3, a_dst3, b3, Wm1, bm1, Wm2, bm2, Wm3, bm3):
    N = x.shape[0]
    npad = EP - E_EDGES
    pad_i = jnp.arange(npad, dtype=jnp.int32)
    src = jnp.concatenate([edge_index[0].astype(jnp.int32), pad_i % N])
    dst = jnp.concatenate([edge_index[1].astype(jnp.int32),
                           N + (pad_i % (NP - N))])
    src2d = src.reshape(EP // K, K)
    dst2d = dst.reshape(EP // K, K)
    xp = jnp.concatenate(
        [x, jnp.zeros((NP - N, x.shape[1]), jnp.float32)], axis=0)

    zn = jnp.zeros((NP,), jnp.float32)
    flat = lambda a: a.reshape(-1)
    cat = lambda a: jnp.concatenate(
        [a[:, :a.shape[1] // 2], a[:, a.shape[1] // 2:]], axis=0)

    h, ts, td, ws, sacc = _tc_stage(xp, W1, a_src1, a_dst1)
    acc_cat, den = _sc_agg_128(cat(h), flat(ts), flat(td), src2d, dst2d,
                               cat(sacc), flat(ws))
    h, ts, td, ws, sacc = _tc_combine_stage(acc_cat, den[:NP], b1,
                                            W2, a_src2, a_dst2)
    acc_cat, den = _sc_agg_64(cat(h), flat(ts), flat(td), src2d, dst2d,
                              cat(sacc), flat(ws))
    W3p = jnp.pad(W3, ((0, 0), (0, 12)))
    a_src3p = jnp.pad(a_src3, (0, 12))
    a_dst3p = jnp.pad(a_dst3, (0, 12))
    b3p = jnp.pad(b3, (0, 12))
    h, ts, td, ws, sacc = _tc_combine_stage(acc_cat, den[:NP], b2,
                                            W3p, a_src3p, a_dst3p)
    acc_cat, den_cat = _sc_agg_16(
        h, flat(ts), flat(td), src2d, dst2d,
        jnp.concatenate([sacc, jnp.zeros((NP, 16), jnp.float32)], axis=0),
        jnp.concatenate([flat(ws), zn]))
    h3 = _tc_combine(acc_cat, den_cat, b3p)[:, :4]
    hrc = _sc_pair(h3, src2d, dst2d).reshape(EP, 8)
    return _edge_mlp(hrc[:E_EDGES], edge_attr,
                     Wm1, bm1, Wm2, bm2, Wm3, bm3)


# speedup vs baseline: 19.7498x; 1.0085x over previous
import functools

import jax
import jax.numpy as jnp
from jax import lax
from jax.experimental import pallas as pl
from jax.experimental.pallas import tpu as pltpu
from jax.experimental.pallas import tpu_sc as plsc


N_NODES = 10000
NP = 10240
E_EDGES = 160000
EP = 163840
PW = EP // 32
K = 128
NCH = PW // K
RPS = NP // 16
MLP_BLK = 8000
TC_BLK = 1280

_SC_MESH = plsc.VectorSubcoreMesh(core_axis_name="c", subcore_axis_name="s",
                                  num_cores=2, num_subcores=16)


def _tc_stage_body(x_ref, w_ref, asrc_ref, adst_ref,
                   h_ref, ts_ref, td_ref, wself_ref, sacc_ref):
    h = x_ref[...] @ w_ref[...]
    ts = h @ asrc_ref[...]
    td = h @ adst_ref[...]
    e = ts + td
    ws = jnp.exp(jnp.maximum(e, 0.2 * e))
    h_ref[...] = h
    ts_ref[...] = ts
    td_ref[...] = td
    wself_ref[...] = ws
    sacc_ref[...] = h * ws


def _tc_stage(x, W, a_src, a_dst):
    cin, c = W.shape
    grid = (NP // TC_BLK,)
    return pl.pallas_call(
        _tc_stage_body,
        grid=grid,
        in_specs=[
            pl.BlockSpec((TC_BLK, cin), lambda i: (i, 0)),
            pl.BlockSpec((cin, c), lambda i: (0, 0)),
            pl.BlockSpec((c, 1), lambda i: (0, 0)),
            pl.BlockSpec((c, 1), lambda i: (0, 0)),
        ],
        out_specs=[
            pl.BlockSpec((TC_BLK, c), lambda i: (i, 0)),
            pl.BlockSpec((TC_BLK, 1), lambda i: (i, 0)),
            pl.BlockSpec((TC_BLK, 1), lambda i: (i, 0)),
            pl.BlockSpec((TC_BLK, 1), lambda i: (i, 0)),
            pl.BlockSpec((TC_BLK, c), lambda i: (i, 0)),
        ],
        out_shape=[
            jax.ShapeDtypeStruct((NP, c), jnp.float32),
            jax.ShapeDtypeStruct((NP, 1), jnp.float32),
            jax.ShapeDtypeStruct((NP, 1), jnp.float32),
            jax.ShapeDtypeStruct((NP, 1), jnp.float32),
            jax.ShapeDtypeStruct((NP, c), jnp.float32),
        ],
    )(x, W, a_src.reshape(-1, 1), a_dst.reshape(-1, 1))


def _tc_combine_stage_body(al_ref, ar_ref, d_ref, b_ref,
                           w_ref, asrc_ref, adst_ref,
                           h_ref, ts_ref, td_ref, wself_ref, sacc_ref):
    acc = jnp.concatenate([al_ref[...], ar_ref[...]], axis=-1)
    x = acc / (d_ref[...] + 1e-16) + b_ref[...]
    h = x @ w_ref[...]
    ts = h @ asrc_ref[...]
    td = h @ adst_ref[...]
    e = ts + td
    ws = jnp.exp(jnp.maximum(e, 0.2 * e))
    h_ref[...] = h
    ts_ref[...] = ts
    td_ref[...] = td
    wself_ref[...] = ws
    sacc_ref[...] = h * ws


def _tc_combine_stage(acc_cat, den, b, W, a_src, a_dst):
    cin, c = W.shape
    hcin = cin // 2
    nb = NP // TC_BLK
    grid = (nb,)
    return pl.pallas_call(
        _tc_combine_stage_body,
        grid=grid,
        in_specs=[
            pl.BlockSpec((TC_BLK, hcin), lambda i: (i, 0)),
            pl.BlockSpec((TC_BLK, hcin), lambda i: (i + NP // TC_BLK, 0)),
            pl.BlockSpec((TC_BLK, 1), lambda i: (i, 0)),
            pl.BlockSpec((1, cin), lambda i: (0, 0)),
            pl.BlockSpec((cin, c), lambda i: (0, 0)),
            pl.BlockSpec((c, 1), lambda i: (0, 0)),
            pl.BlockSpec((c, 1), lambda i: (0, 0)),
        ],
        out_specs=[
            pl.BlockSpec((TC_BLK, c), lambda i: (i, 0)),
            pl.BlockSpec((TC_BLK, 1), lambda i: (i, 0)),
            pl.BlockSpec((TC_BLK, 1), lambda i: (i, 0)),
            pl.BlockSpec((TC_BLK, 1), lambda i: (i, 0)),
            pl.BlockSpec((TC_BLK, c), lambda i: (i, 0)),
        ],
        out_shape=[
            jax.ShapeDtypeStruct((NP, c), jnp.float32),
            jax.ShapeDtypeStruct((NP, 1), jnp.float32),
            jax.ShapeDtypeStruct((NP, 1), jnp.float32),
            jax.ShapeDtypeStruct((NP, 1), jnp.float32),
            jax.ShapeDtypeStruct((NP, c), jnp.float32),
        ],
    )(acc_cat, acc_cat, den.reshape(-1, 1), b.reshape(1, -1), W,
      a_src.reshape(-1, 1), a_dst.reshape(-1, 1))


def _tc_combine_body(a0_ref, a1_ref, d0_ref, d1_ref, b_ref, out_ref):
    out_ref[...] = ((a0_ref[...] + a1_ref[...])
                    / (d0_ref[...] + d1_ref[...] + 1e-16) + b_ref[...])


def _tc_combine(acc_cat, den_cat, b):
    c = acc_cat.shape[1]
    grid = (NP // TC_BLK,)
    return pl.pallas_call(
        _tc_combine_body,
        grid=grid,
        in_specs=[
            pl.BlockSpec((TC_BLK, c), lambda i: (i, 0)),
            pl.BlockSpec((TC_BLK, c), lambda i: (i + NP // TC_BLK, 0)),
            pl.BlockSpec((TC_BLK, 1), lambda i: (i, 0)),
            pl.BlockSpec((TC_BLK, 1), lambda i: (i + NP // TC_BLK, 0)),
            pl.BlockSpec((1, c), lambda i: (0, 0)),
        ],
        out_specs=pl.BlockSpec((TC_BLK, c), lambda i: (i, 0)),
        out_shape=jax.ShapeDtypeStruct((NP, c), jnp.float32),
    )(acc_cat, acc_cat, den_cat.reshape(-1, 1), den_cat.reshape(-1, 1),
      b.reshape(1, -1))


def _iota16():
    return lax.iota(jnp.int32, 16)


NCH2 = EP // 16 // K


def _make_sc_aggregate(c):
    hc = c // 2

    def body(hcat_hbm, ts_hbm, td_hbm, src_hbm, dst_hbm,
             init_hbm, wself_hbm,
             acc_out, den_out,
             ts_v, td_v, src_v, dst_v, w_v, rows0, rows1,
             acc_sh, den_sh, sem0, sem1):
        cid = lax.axis_index("c")
        sid = lax.axis_index("s")
        r0 = sid * RPS
        pltpu.sync_copy(init_hbm.at[pl.ds(cid * NP + r0, RPS)],
                        acc_sh.at[pl.ds(r0, RPS)])
        pltpu.sync_copy(wself_hbm.at[pl.ds(r0, RPS)],
                        den_sh.at[pl.ds(r0, RPS)])

        pltpu.sync_copy(ts_hbm, ts_v)
        pltpu.sync_copy(td_hbm, td_v)
        pltpu.sync_copy(src_hbm.at[pl.ds(sid * NCH2, NCH2)], src_v)
        pltpu.sync_copy(dst_hbm.at[pl.ds(sid * NCH2, NCH2)], dst_v)

        off = jnp.full((16,), cid * NP, jnp.int32)

        def wchunk(i, _):
            for j in range(K // 16):
                sl = pl.ds(j * 16, 16)
                sidx = src_v[i, sl]
                didx = dst_v[i, sl]
                e = (plsc.load_gather(ts_v, [sidx])
                     + plsc.load_gather(td_v, [didx]))
                w_v[i, sl] = jnp.exp(jnp.maximum(e, 0.2 * e))
                src_v[i, sl] = sidx + off
            return 0

        lax.fori_loop(0, NCH2, wchunk, 0)
        plsc.subcore_barrier()

        pltpu.async_copy(hcat_hbm.at[src_v.at[0]], rows0, sem0)

        def chunk(i, cur, csem, pf, nxt, nsem):
            pltpu.make_async_copy(hcat_hbm.at[src_v.at[i]], cur, csem).wait()

            @pl.when(pf < NCH2)
            def _():
                pltpu.async_copy(hcat_hbm.at[src_v.at[pf]], nxt, nsem)

            @plsc.parallel_loop(0, K, unroll=8)
            def mul_row(e2):
                wb = plsc.load_gather(
                    w_v, [jnp.full((16,), i, jnp.int32),
                          jnp.full((16,), e2, jnp.int32)])
                for jj in range(hc // 16):
                    sl2 = pl.ds(jj * 16, 16)
                    cur[e2, sl2] = cur[e2, sl2] * wb
            pltpu.sync_copy(cur, acc_sh.at[dst_v.at[i]], add=True)
            pltpu.sync_copy(w_v.at[i], den_sh.at[dst_v.at[i]], add=True)

        def pair(p, _):
            i0 = p * 2
            chunk(i0, rows0, sem0, i0 + 1, rows1, sem1)
            chunk(i0 + 1, rows1, sem1, i0 + 2, rows0, sem0)
            return 0

        lax.fori_loop(0, NCH2 // 2, pair, 0)
        plsc.subcore_barrier()
        pltpu.sync_copy(acc_sh.at[pl.ds(r0, RPS)],
                        acc_out.at[pl.ds(cid * NP + r0, RPS)])
        pltpu.sync_copy(den_sh.at[pl.ds(r0, RPS)],
                        den_out.at[pl.ds(cid * NP + r0, RPS)])

    return pl.kernel(
        body,
        out_type=[
            jax.ShapeDtypeStruct((2 * NP, hc), jnp.float32),
            jax.ShapeDtypeStruct((2 * NP,), jnp.float32),
        ],
        mesh=_SC_MESH,
        compiler_params=pltpu.CompilerParams(needs_layout_passes=False, use_tc_tiling_on_sc=False),
        scratch_types=[
            pltpu.VMEM((NP,), jnp.float32),
            pltpu.VMEM((NP,), jnp.float32),
            pltpu.VMEM((NCH2, K), jnp.int32),
            pltpu.VMEM((NCH2, K), jnp.int32),
            pltpu.VMEM((NCH2, K), jnp.float32),
            pltpu.VMEM((K, hc), jnp.float32),
            pltpu.VMEM((K, hc), jnp.float32),
            pltpu.VMEM_SHARED((NP, hc), jnp.float32),
            pltpu.VMEM_SHARED((NP,), jnp.float32),
            pltpu.SemaphoreType.DMA,
            pltpu.SemaphoreType.DMA,
        ],
    )


def _make_sc_aggregate_es(c):

    def body(h_hbm, ts_hbm, td_hbm, src_hbm, dst_hbm,
             initcat_hbm, wscat_hbm,
             acc_out, den_out,
             ts_v, td_v, src_v, dst_v, w_v, rows0, rows1,
             acc_sh, den_sh, sem0, sem1):
        cid = lax.axis_index("c")
        sid = lax.axis_index("s")
        wid = cid * 16 + sid
        r0 = sid * RPS
        pltpu.sync_copy(initcat_hbm.at[pl.ds(cid * NP + r0, RPS)],
                        acc_sh.at[pl.ds(r0, RPS)])
        pltpu.sync_copy(wscat_hbm.at[pl.ds(cid * NP + r0, RPS)],
                        den_sh.at[pl.ds(r0, RPS)])

        pltpu.sync_copy(ts_hbm, ts_v)
        pltpu.sync_copy(td_hbm, td_v)
        pltpu.sync_copy(src_hbm.at[pl.ds(wid * NCH, NCH)], src_v)
        pltpu.sync_copy(dst_hbm.at[pl.ds(wid * NCH, NCH)], dst_v)

        def wchunk(i, _):
            for j in range(K // 16):
                sl = pl.ds(j * 16, 16)
                sidx = src_v[i, sl]
                didx = dst_v[i, sl]
                e = (plsc.load_gather(ts_v, [sidx])
                     + plsc.load_gather(td_v, [didx]))
                w_v[i, sl] = jnp.exp(jnp.maximum(e, 0.2 * e))
            return 0

        lax.fori_loop(0, NCH, wchunk, 0)
        plsc.subcore_barrier()

        pltpu.async_copy(h_hbm.at[src_v.at[0]], rows0, sem0)

        def chunk(i, cur, csem, pf, nxt, nsem):
            pltpu.make_async_copy(h_hbm.at[src_v.at[i]], cur, csem).wait()

            @pl.when(pf < NCH)
            def _():
                pltpu.async_copy(h_hbm.at[src_v.at[pf]], nxt, nsem)

            @plsc.parallel_loop(0, K, unroll=8)
            def mul_row(e2):
                wb = plsc.load_gather(
                    w_v, [jnp.full((16,), i, jnp.int32),
                          jnp.full((16,), e2, jnp.int32)])
                for jj in range(c // 16):
                    sl2 = pl.ds(jj * 16, 16)
                    cur[e2, sl2] = cur[e2, sl2] * wb
            pltpu.sync_copy(cur, acc_sh.at[dst_v.at[i]], add=True)
            pltpu.sync_copy(w_v.at[i], den_sh.at[dst_v.at[i]], add=True)

        def pair(p, _):
            i0 = p * 2
            chunk(i0, rows0, sem0, i0 + 1, rows1, sem1)
            chunk(i0 + 1, rows1, sem1, i0 + 2, rows0, sem0)
            return 0

        lax.fori_loop(0, NCH // 2, pair, 0)
        plsc.subcore_barrier()
        pltpu.sync_copy(acc_sh.at[pl.ds(r0, RPS)],
                        acc_out.at[pl.ds(cid * NP + r0, RPS)])
        pltpu.sync_copy(den_sh.at[pl.ds(r0, RPS)],
                        den_out.at[pl.ds(cid * NP + r0, RPS)])

    return pl.kernel(
        body,
        out_type=[
            jax.ShapeDtypeStruct((2 * NP, c), jnp.float32),
            jax.ShapeDtypeStruct((2 * NP,), jnp.float32),
        ],
        mesh=_SC_MESH,
        compiler_params=pltpu.CompilerParams(needs_layout_passes=False, use_tc_tiling_on_sc=False),
        scratch_types=[
            pltpu.VMEM((NP,), jnp.float32),
            pltpu.VMEM((NP,), jnp.float32),
            pltpu.VMEM((NCH, K), jnp.int32),
            pltpu.VMEM((NCH, K), jnp.int32),
            pltpu.VMEM((NCH, K), jnp.float32),
            pltpu.VMEM((K, c), jnp.float32),
            pltpu.VMEM((K, c), jnp.float32),
            pltpu.VMEM_SHARED((NP, c), jnp.float32),
            pltpu.VMEM_SHARED((NP,), jnp.float32),
            pltpu.SemaphoreType.DMA,
            pltpu.SemaphoreType.DMA,
        ],
    )


def _make_sc_pair_gather():
    def body(h_hbm, src_hbm, dst_hbm, hrc_out,
             h3_v, src_v, dst_v, rbuf):
        cid = lax.axis_index("c")
        sid = lax.axis_index("s")
        wid = cid * 16 + sid
        pltpu.sync_copy(h_hbm, h3_v)
        pltpu.sync_copy(src_hbm.at[pl.ds(wid * NCH, NCH)], src_v)
        pltpu.sync_copy(dst_hbm.at[pl.ds(wid * NCH, NCH)], dst_v)

        lanes = _iota16()
        e2 = lanes >> 3
        sel = (lanes >> 2) & 1
        f4 = lanes & 3

        def chunk(i, _):
            irow = jnp.full((16,), i, jnp.int32)

            @plsc.parallel_loop(0, K // 2, unroll=4)
            def grp(g):
                eloc = g * 2 + e2
                nr = plsc.load_gather(src_v, [irow, eloc])
                nc = plsc.load_gather(dst_v, [irow, eloc])
                node = jnp.where(sel == 1, nc, nr)
                rbuf[pl.ds(g * 16, 16)] = plsc.load_gather(h3_v, [node, f4])

            base = (wid * PW + i * K) * 8
            pltpu.sync_copy(rbuf, hrc_out.at[pl.ds(base, K * 8)])
            return 0

        lax.fori_loop(0, NCH, chunk, 0)

    return pl.kernel(
        body,
        out_type=jax.ShapeDtypeStruct((EP * 8,), jnp.float32),
        mesh=_SC_MESH,
        compiler_params=pltpu.CompilerParams(needs_layout_passes=False, use_tc_tiling_on_sc=False),
        scratch_types=[
            pltpu.VMEM((NP, 4), jnp.float32),
            pltpu.VMEM((NCH, K), jnp.int32),
            pltpu.VMEM((NCH, K), jnp.int32),
            pltpu.VMEM((K * 8,), jnp.float32),
        ],
    )


_sc_agg_128 = _make_sc_aggregate(128)
_sc_agg_64 = _make_sc_aggregate(64)
_sc_agg_16 = _make_sc_aggregate_es(16)
_sc_pair = _make_sc_pair_gather()


def _mlp_body(hrc_ref, ea_ref, w1p_ref, w1e_ref, b1_ref,
              w2_ref, b2_ref, w3_ref, b3_ref, out_ref):
    z = (jnp.dot(hrc_ref[...].astype(jnp.bfloat16), w1p_ref[...],
                 preferred_element_type=jnp.float32)
         + jnp.dot(ea_ref[...].astype(jnp.bfloat16), w1e_ref[...],
                   preferred_element_type=jnp.float32) + b1_ref[...])
    z = jnp.maximum(z, 0.0)
    z = jnp.dot(z.astype(jnp.bfloat16), w2_ref[...],
                preferred_element_type=jnp.float32) + b2_ref[...]
    z = jnp.maximum(z, 0.0)
    z = jnp.dot(z.astype(jnp.bfloat16), w3_ref[...],
                preferred_element_type=jnp.float32) + b3_ref[...]
    out_ref[...] = ea_ref[...] + z


def _edge_mlp(hrc, ea, Wm1, bm1, Wm2, bm2, Wm3, bm3):
    E = ea.shape[0]
    hid = Wm1.shape[1]
    edim = Wm3.shape[1]
    grid = (E // MLP_BLK,)
    return pl.pallas_call(
        _mlp_body,
        grid=grid,
        in_specs=[
            pl.BlockSpec((MLP_BLK, 8), lambda i: (i, 0)),
            pl.BlockSpec((MLP_BLK, 16), lambda i: (i, 0)),
            pl.BlockSpec((8, hid), lambda i: (0, 0)),
            pl.BlockSpec((16, hid), lambda i: (0, 0)),
            pl.BlockSpec((1, hid), lambda i: (0, 0)),
            pl.BlockSpec((hid, hid), lambda i: (0, 0)),
            pl.BlockSpec((1, hid), lambda i: (0, 0)),
            pl.BlockSpec((hid, edim), lambda i: (0, 0)),
            pl.BlockSpec((1, edim), lambda i: (0, 0)),
        ],
        out_specs=pl.BlockSpec((MLP_BLK, edim), lambda i: (i, 0)),
        out_shape=jax.ShapeDtypeStruct((E, edim), jnp.float32),
    )(hrc, ea,
      Wm1[:8].astype(jnp.bfloat16), Wm1[8:].astype(jnp.bfloat16),
      bm1.reshape(1, -1),
      Wm2.astype(jnp.bfloat16), bm2.reshape(1, -1),
      Wm3.astype(jnp.bfloat16), bm3.reshape(1, -1))


def kernel(x, edge_index, edge_attr, W1, a_src1, a_dst1, b1, W2, a_src2, a_dst2, b2, W3, a_src3, a_dst3, b3, Wm1, bm1, Wm2, bm2, Wm3, bm3):
    N = x.shape[0]
    npad = EP - E_EDGES
    pad_i = jnp.arange(npad, dtype=jnp.int32)
    src = jnp.concatenate([edge_index[0].astype(jnp.int32), pad_i % N])
    dst = jnp.concatenate([edge_index[1].astype(jnp.int32),
                           N + (pad_i % (NP - N))])
    src2d = src.reshape(EP // K, K)
    dst2d = dst.reshape(EP // K, K)
    xp = jnp.concatenate(
        [x, jnp.zeros((NP - N, x.shape[1]), jnp.float32)], axis=0)

    zn = jnp.zeros((NP,), jnp.float32)
    flat = lambda a: a.reshape(-1)
    cat = lambda a: jnp.concatenate(
        [a[:, :a.shape[1] // 2], a[:, a.shape[1] // 2:]], axis=0)

    h, ts, td, ws, sacc = _tc_stage(xp, W1, a_src1, a_dst1)
    acc_cat, den = _sc_agg_128(cat(h), flat(ts), flat(td), src2d, dst2d,
                               cat(sacc), flat(ws))
    h, ts, td, ws, sacc = _tc_combine_stage(acc_cat, den[:NP], b1,
                                            W2, a_src2, a_dst2)
    acc_cat, den = _sc_agg_64(cat(h), flat(ts), flat(td), src2d, dst2d,
                              cat(sacc), flat(ws))
    W3p = jnp.pad(W3, ((0, 0), (0, 12)))
    a_src3p = jnp.pad(a_src3, (0, 12))
    a_dst3p = jnp.pad(a_dst3, (0, 12))
    b3p = jnp.pad(b3, (0, 12))
    h, ts, td, ws, sacc = _tc_combine_stage(acc_cat, den[:NP], b2,
                                            W3p, a_src3p, a_dst3p)
    acc_cat, den_cat = _sc_agg_16(
        h, flat(ts), flat(td), src2d, dst2d,
        jnp.concatenate([sacc, jnp.zeros((NP, 16), jnp.float32)], axis=0),
        jnp.concatenate([flat(ws), zn]))
    h3 = _tc_combine(acc_cat, den_cat, b3p)[:, :4]
    hrc = _sc_pair(h3, src2d, dst2d).reshape(EP, 8)
    return _edge_mlp(hrc[:E_EDGES], edge_attr,
                     Wm1, bm1, Wm2, bm2, Wm3, bm3)
